# baseline (speedup 1.0000x reference)
import functools

import jax
import jax.numpy as jnp
from jax import lax
from jax.experimental import pallas as pl
from jax.experimental.pallas import tpu as pltpu
from jax.experimental.pallas import tpu_sc as plsc

B = 10
NP = 1000
H = 128
E = 320000
EPG = E // B
NC, NS = 2, 16
GPC = B // NC
EPT = EPG // NS
CHUNK = 80
NCHUNK = EPT // CHUNK
VPC = CHUNK // 16
CT = 8
ACC = CT * NP * 128
TQ = ACC // NS
NCK = 8
CK = TQ // NCK


def _flat_addr(sl, dl):
    return (jnp.right_shift(sl, 7) * (NP * 128)
            + dl * 128 + jnp.bitwise_and(sl, 127))


def _adj_body(ei_hbm, ew_hbm, adj_hbm,
              src_v, dst_v, ew_v, idx_v, zero_v, bounce0_v, bounce1_v,
              acc_sh, zsem, csem0, csem1, ssem):
    cid = lax.axis_index("c")
    sid = lax.axis_index("s")

    def zfill(i, c):
        zero_v[pl.ds(i * 16, 16)] = jnp.zeros((16,), jnp.float32)
        return c
    lax.fori_loop(0, CK // 16, zfill, 0)

    def per_graph(g, carry):
        b = cid * GPC + g
        base_w = sid * TQ

        @pl.when(g == 0)
        def _():
            def zero_chunk(k, c):
                pltpu.async_copy(zero_v,
                                 acc_sh.at[pl.ds(base_w + k * CK, CK)], zsem)
                return c
            lax.fori_loop(0, NCK, zero_chunk, 0)

        base_e = b * EPG + sid * EPT
        pltpu.sync_copy(ei_hbm.at[pl.ds(base_e, EPT)], src_v)
        pltpu.sync_copy(ei_hbm.at[pl.ds(E + base_e, EPT)], dst_v)
        pltpu.sync_copy(ew_hbm.at[pl.ds(base_e, EPT)], ew_v)
        nb = b * NP

        def mkidx(i, c):
            j = i // VPC
            col = (i % VPC) * 16
            sl = src_v[pl.ds(i * 16, 16)] - nb
            dl = dst_v[pl.ds(i * 16, 16)] - nb
            idx_v[j, pl.ds(col, 16)] = _flat_addr(sl, dl)
            return c
        lax.fori_loop(0, EPT // 16, mkidx, 0)

        def zdrain(k, c):
            pltpu.make_async_copy(
                zero_v, acc_sh.at[pl.ds(base_w, CK)], zsem).wait()
            return c
        lax.fori_loop(0, NCK, zdrain, 0)
        plsc.subcore_barrier()

        descs = []
        for j in range(NCHUNK):
            descs.append(pltpu.async_copy(
                ew_v.at[pl.ds(j * CHUNK, CHUNK)],
                acc_sh.at[idx_v.at[j]], ssem, add=True))
        for dsc in descs:
            dsc.wait()
        plsc.subcore_barrier()

        for k in range(NCK):
            bv = bounce0_v if k % 2 == 0 else bounce1_v
            sem = csem0 if k % 2 == 0 else csem1
            off = base_w + k * CK
            if k >= 2:
                pltpu.make_async_copy(
                    bv, adj_hbm.at[pl.ds(b * ACC + off, CK)], sem).wait()
            pltpu.sync_copy(acc_sh.at[pl.ds(off, CK)], bv)
            pltpu.async_copy(bv, adj_hbm.at[pl.ds(b * ACC + off, CK)], sem)

            @pl.when(g < GPC - 1)
            def _():
                pltpu.async_copy(zero_v, acc_sh.at[pl.ds(off, CK)], zsem)
        for k in (NCK - 2, NCK - 1):
            bv = bounce0_v if k % 2 == 0 else bounce1_v
            sem = csem0 if k % 2 == 0 else csem1
            off = base_w + k * CK
            pltpu.make_async_copy(
                bv, adj_hbm.at[pl.ds(b * ACC + off, CK)], sem).wait()
        plsc.subcore_barrier()
        return carry
    lax.fori_loop(0, GPC, per_graph, 0)


@functools.cache
def _build_adj_kernel():
    return pl.kernel(
        _adj_body,
        out_type=jax.ShapeDtypeStruct((B * ACC,), jnp.float32),
        mesh=plsc.VectorSubcoreMesh(core_axis_name="c", subcore_axis_name="s",
                                    num_cores=NC, num_subcores=NS),
        scratch_types=[
            pltpu.VMEM((EPT,), jnp.int32),
            pltpu.VMEM((EPT,), jnp.int32),
            pltpu.VMEM((EPT,), jnp.float32),
            pltpu.VMEM((NCHUNK, CHUNK), jnp.int32),
            pltpu.VMEM((CK,), jnp.float32),
            pltpu.VMEM((CK,), jnp.float32),
            pltpu.VMEM((CK,), jnp.float32),
            pltpu.VMEM_SHARED((ACC,), jnp.float32),
            pltpu.SemaphoreType.DMA,
            pltpu.SemaphoreType.DMA,
            pltpu.SemaphoreType.DMA,
            pltpu.SemaphoreType.DMA,
        ],
    )


def _adj_matmul(adj_ref, m):
    mp = jnp.concatenate(
        [m, jnp.zeros((CT * 128 - NP, m.shape[1]), m.dtype)],
        axis=0).astype(jnp.bfloat16)
    acc = None
    for tc in range(CT):
        part = jnp.dot(adj_ref[pl.ds(tc * NP, NP), :],
                       mp[tc * 128:(tc + 1) * 128],
                       preferred_element_type=jnp.float32)
        acc = part if acc is None else acc + part
    return acc


def _conv1_body(x_ref, adj_ref, wpre_ref, bpre_ref, wrel_ref, brel_ref,
                wroot_ref, h1_ref, abf_ref, sum_ref, sq_ref, d_ref):
    xb = x_ref[0]
    h0 = jnp.dot(xb, wpre_ref[...],
                 preferred_element_type=jnp.float32) + bpre_ref[...]
    mp = jnp.concatenate(
        [h0, jnp.zeros((CT * 128 - NP, H), h0.dtype)],
        axis=0).astype(jnp.bfloat16)
    agg = None
    dparts = []
    for tc in range(CT):
        a32 = adj_ref[pl.ds(tc * NP, NP), :]
        abf = a32.astype(jnp.bfloat16)
        abf_ref[pl.ds(tc * NP, NP), :] = abf
        part = jnp.dot(abf, mp[tc * 128:(tc + 1) * 128],
                       preferred_element_type=jnp.float32)
        agg = part if agg is None else agg + part
        dparts.append(jnp.sum(a32, axis=0))
    h1 = (jnp.dot(agg, wrel_ref[...], preferred_element_type=jnp.float32)
          + jnp.dot(h0, wroot_ref[...], preferred_element_type=jnp.float32)
          + brel_ref[...])
    h1_ref[0] = h1
    sum_ref[0] = jnp.sum(h1, axis=0, keepdims=True)
    sq_ref[0] = jnp.sum(h1 * h1, axis=0, keepdims=True)
    d_ref[0] = jnp.concatenate(dparts)[:NP].reshape(1, NP)


def _bn_stats(sum_ref, sq_ref):
    mu = jnp.sum(sum_ref[...], axis=0) / (B * NP)
    ex2 = jnp.sum(sq_ref[...], axis=0) / (B * NP)
    var = ex2 - mu * mu
    return mu, lax.rsqrt(var + 1e-5)


def _conv2_body(h1_ref, adj_ref, sum_ref, sq_ref, g_ref, be_ref, wrel_ref,
                brel_ref, wroot_ref, h2_ref, sum2_ref, sq2_ref):
    mu, inv = _bn_stats(sum_ref, sq_ref)
    hn = jnp.maximum((h1_ref[0] - mu) * inv * g_ref[...] + be_ref[...], 0.0)
    agg = _adj_matmul(adj_ref, hn)
    h2 = (jnp.dot(agg, wrel_ref[...], preferred_element_type=jnp.float32)
          + jnp.dot(hn, wroot_ref[...], preferred_element_type=jnp.float32)
          + brel_ref[...])
    h2_ref[0] = h2
    sum2_ref[0] = jnp.sum(h2, axis=0, keepdims=True)
    sq2_ref[0] = jnp.sum(h2 * h2, axis=0, keepdims=True)


def _pool_body(h2_ref, adj_ref, d_ref, sum_ref, sq_ref, g_ref, be_ref,
               wpool_ref, bpool_ref, wpt_ref, wpb_ref, bpost_ref,
               out_ref, mc_ref, ol_ref):
    b = pl.program_id(0)
    mu, inv = _bn_stats(sum_ref, sq_ref)
    hn = jnp.maximum((h2_ref[0] - mu) * inv * g_ref[...] + be_ref[...], 0.0)
    z = jnp.dot(hn, wpool_ref[...],
                preferred_element_type=jnp.float32) + bpool_ref[...]
    m = jnp.max(z, axis=1, keepdims=True)
    ez = jnp.exp(z - m)
    s = ez / jnp.sum(ez, axis=1, keepdims=True)
    p0 = jnp.maximum(jnp.sum(s[:, 0:1] * hn, axis=0, keepdims=True), 0.0)
    p1 = jnp.maximum(jnp.sum(s[:, 1:2] * hn, axis=0, keepdims=True), 0.0)
    out = (jnp.dot(p0, wpt_ref[...], preferred_element_type=jnp.float32)
           + jnp.dot(p1, wpb_ref[...], preferred_element_type=jnp.float32)
           + bpost_ref[...])
    out_ref[0] = out
    w = _adj_matmul(adj_ref, s)
    num = jnp.sum(w * s)
    den = jnp.sum(d_ref[0, 0] * jnp.sum(s * s, axis=1))
    s0 = s[:, 0:1]
    s1 = s[:, 1:2]
    s00 = jnp.sum(s0 * s0)
    s01 = jnp.sum(s0 * s1)
    s11 = jnp.sum(s1 * s1)
    fro = jnp.sqrt(s00 * s00 + 2.0 * s01 * s01 + s11 * s11)
    c = 1.0 / jnp.sqrt(jnp.float32(2.0))
    a00 = s00 / fro - c
    a01 = s01 / fro
    a11 = s11 / fro - c
    mc = -(num / den) * (1.0 / B)
    ob = jnp.sqrt(a00 * a00 + 2.0 * a01 * a01 + a11 * a11) * (1.0 / B)

    @pl.when(b == 0)
    def _():
        mc_ref[...] = jnp.zeros((1, 1), jnp.float32)
        ol_ref[...] = jnp.zeros((1, 1), jnp.float32)
    mc_ref[...] += jnp.reshape(mc, (1, 1))
    ol_ref[...] += jnp.reshape(ob, (1, 1))


def _full(shape):
    return pl.BlockSpec(shape, lambda b: tuple(0 for _ in shape))


_conv1 = pl.pallas_call(
    _conv1_body,
    grid=(B,),
    in_specs=[
        pl.BlockSpec((1, NP, H), lambda b: (b, 0, 0)),
        pl.BlockSpec((CT * NP, 128), lambda b: (b, 0)),
        _full((H, H)), _full((1, H)), _full((H, H)), _full((1, H)),
        _full((H, H)),
    ],
    out_specs=[
        pl.BlockSpec((1, NP, H), lambda b: (b, 0, 0)),
        pl.BlockSpec((CT * NP, 128), lambda b: (b, 0)),
        pl.BlockSpec((1, 1, H), lambda b: (b, 0, 0)),
        pl.BlockSpec((1, 1, H), lambda b: (b, 0, 0)),
        pl.BlockSpec((1, 1, NP), lambda b: (b, 0, 0)),
    ],
    out_shape=[
        jax.ShapeDtypeStruct((B, NP, H), jnp.float32),
        jax.ShapeDtypeStruct((B * CT * NP, 128), jnp.bfloat16),
        jax.ShapeDtypeStruct((B, 1, H), jnp.float32),
        jax.ShapeDtypeStruct((B, 1, H), jnp.float32),
        jax.ShapeDtypeStruct((B, 1, NP), jnp.float32),
    ],
)

_conv2 = pl.pallas_call(
    _conv2_body,
    grid=(B,),
    in_specs=[
        pl.BlockSpec((1, NP, H), lambda b: (b, 0, 0)),
        pl.BlockSpec((CT * NP, 128), lambda b: (b, 0)),
        pl.BlockSpec((B, 1, H), lambda b: (0, 0, 0)),
        pl.BlockSpec((B, 1, H), lambda b: (0, 0, 0)),
        _full((1, H)), _full((1, H)), _full((H, H)), _full((1, H)),
        _full((H, H)),
    ],
    out_specs=[
        pl.BlockSpec((1, NP, H), lambda b: (b, 0, 0)),
        pl.BlockSpec((1, 1, H), lambda b: (b, 0, 0)),
        pl.BlockSpec((1, 1, H), lambda b: (b, 0, 0)),
    ],
    out_shape=[
        jax.ShapeDtypeStruct((B, NP, H), jnp.float32),
        jax.ShapeDtypeStruct((B, 1, H), jnp.float32),
        jax.ShapeDtypeStruct((B, 1, H), jnp.float32),
    ],
)

_pool = pl.pallas_call(
    _pool_body,
    grid=(B,),
    in_specs=[
        pl.BlockSpec((1, NP, H), lambda b: (b, 0, 0)),
        pl.BlockSpec((CT * NP, 128), lambda b: (b, 0)),
        pl.BlockSpec((1, 1, NP), lambda b: (b, 0, 0)),
        pl.BlockSpec((B, 1, H), lambda b: (0, 0, 0)),
        pl.BlockSpec((B, 1, H), lambda b: (0, 0, 0)),
        _full((1, H)), _full((1, H)), _full((H, 2)), _full((1, 2)),
        _full((H, 2)), _full((H, 2)), _full((1, 2)),
    ],
    out_specs=[
        pl.BlockSpec((1, 1, 2), lambda b: (b, 0, 0)),
        pl.BlockSpec((1, 1), lambda b: (0, 0)),
        pl.BlockSpec((1, 1), lambda b: (0, 0)),
    ],
    out_shape=[
        jax.ShapeDtypeStruct((B, 1, 2), jnp.float32),
        jax.ShapeDtypeStruct((1, 1), jnp.float32),
        jax.ShapeDtypeStruct((1, 1), jnp.float32),
    ],
)


def kernel(x, edge_index, edge_feature, batch, W_pre, b_pre, Wrel1, brel1,
           Wroot1, g1, be1, Wrel2, brel2, Wroot2, g2, be2, W_pool, b_pool,
           W_post, b_post):
    ew = edge_feature.reshape(E)
    adj = _build_adj_kernel()(edge_index.reshape(2 * E),
                              ew).reshape(B * CT * NP, 128)
    x3 = x.reshape(B, NP, H)
    h1, abf, s1, q1, d = _conv1(x3, adj, W_pre, b_pre.reshape(1, H), Wrel1,
                                brel1.reshape(1, H), Wroot1)
    h2, s2, q2 = _conv2(h1, abf, s1, q1, g1.reshape(1, H), be1.reshape(1, H),
                        Wrel2, brel2.reshape(1, H), Wroot2)
    out3, mc, ol = _pool(h2, abf, d, s2, q2, g2.reshape(1, H),
                         be2.reshape(1, H), W_pool, b_pool.reshape(1, 2),
                         W_post[:H], W_post[H:], b_post.reshape(1, 2))
    return out3.reshape(B, 2), mc[0, 0], ol[0, 0]


# speedup vs baseline: 1.0141x; 1.0141x over previous
import functools

import jax
import jax.numpy as jnp
from jax import lax
from jax.experimental import pallas as pl
from jax.experimental.pallas import tpu as pltpu
from jax.experimental.pallas import tpu_sc as plsc

B = 10
NP = 1000
H = 128
E = 320000
EPG = E // B
NC, NS = 2, 16
GPC = B // NC
EPT = EPG // NS
CHUNK = 80
NCHUNK = EPT // CHUNK
VPC = CHUNK // 16
CT = 8
ACC = CT * NP * 128
TQ = ACC // NS
NCK = 8
CK = TQ // NCK


def _flat_addr(sl, dl):
    return (jnp.right_shift(sl, 7) * (NP * 128)
            + dl * 128 + jnp.bitwise_and(sl, 127))


def _adj_body(ei_hbm, ew_hbm, adj_hbm,
              src_v, dst_v, ew_v, idx_v, zero_v, bounce0_v, bounce1_v,
              bounce2_v, acc_sh, zsem, csem0, csem1, csem2, ssem):
    cid = lax.axis_index("c")
    sid = lax.axis_index("s")

    def zfill(i, c):
        zero_v[pl.ds(i * 16, 16)] = jnp.zeros((16,), jnp.float32)
        return c
    lax.fori_loop(0, CK // 16, zfill, 0)

    def per_graph(g, carry):
        b = cid * GPC + g
        base_w = sid * TQ

        @pl.when(g == 0)
        def _():
            def zero_chunk(k, c):
                pltpu.async_copy(zero_v,
                                 acc_sh.at[pl.ds(base_w + k * CK, CK)], zsem)
                return c
            lax.fori_loop(0, NCK, zero_chunk, 0)

        base_e = b * EPG + sid * EPT
        pltpu.sync_copy(ei_hbm.at[pl.ds(base_e, EPT)], src_v)
        pltpu.sync_copy(ei_hbm.at[pl.ds(E + base_e, EPT)], dst_v)
        pltpu.sync_copy(ew_hbm.at[pl.ds(base_e, EPT)], ew_v)
        nb = b * NP

        def mkidx(i, c):
            j = i // VPC
            col = (i % VPC) * 16
            sl = src_v[pl.ds(i * 16, 16)] - nb
            dl = dst_v[pl.ds(i * 16, 16)] - nb
            idx_v[j, pl.ds(col, 16)] = _flat_addr(sl, dl)
            return c
        lax.fori_loop(0, EPT // 16, mkidx, 0)

        def zdrain(k, c):
            pltpu.make_async_copy(
                zero_v, acc_sh.at[pl.ds(base_w, CK)], zsem).wait()
            return c
        lax.fori_loop(0, NCK, zdrain, 0)
        plsc.subcore_barrier()

        descs = []
        for j in range(NCHUNK):
            descs.append(pltpu.async_copy(
                ew_v.at[pl.ds(j * CHUNK, CHUNK)],
                acc_sh.at[idx_v.at[j]], ssem, add=True))
        for dsc in descs:
            dsc.wait()
        plsc.subcore_barrier()

        bvs = (bounce0_v, bounce1_v, bounce2_v)
        sems = (csem0, csem1, csem2)

        def fire_zero(off):
            @pl.when(g < GPC - 1)
            def _():
                pltpu.async_copy(zero_v, acc_sh.at[pl.ds(off, CK)], zsem)

        h1d = [None, None, None]
        h2d = [None, None, None]
        for k in range(NCK):
            i = k % 3
            if h2d[i] is not None:
                h2d[i].wait()
            h1d[i] = pltpu.async_copy(
                acc_sh.at[pl.ds(base_w + k * CK, CK)], bvs[i], sems[i])
            if k >= 1:
                j = (k - 1) % 3
                h1d[j].wait()
                h2d[j] = pltpu.async_copy(
                    bvs[j],
                    adj_hbm.at[pl.ds(b * ACC + base_w + (k - 1) * CK, CK)],
                    sems[j])
                fire_zero(base_w + (k - 1) * CK)
        j = (NCK - 1) % 3
        h1d[j].wait()
        h2d[j] = pltpu.async_copy(
            bvs[j], adj_hbm.at[pl.ds(b * ACC + base_w + (NCK - 1) * CK, CK)],
            sems[j])
        fire_zero(base_w + (NCK - 1) * CK)
        for j in range(3):
            h2d[j].wait()
        plsc.subcore_barrier()
        return carry
    lax.fori_loop(0, GPC, per_graph, 0)


@functools.cache
def _build_adj_kernel():
    return pl.kernel(
        _adj_body,
        out_type=jax.ShapeDtypeStruct((B * ACC,), jnp.float32),
        mesh=plsc.VectorSubcoreMesh(core_axis_name="c", subcore_axis_name="s",
                                    num_cores=NC, num_subcores=NS),
        scratch_types=[
            pltpu.VMEM((EPT,), jnp.int32),
            pltpu.VMEM((EPT,), jnp.int32),
            pltpu.VMEM((EPT,), jnp.float32),
            pltpu.VMEM((NCHUNK, CHUNK), jnp.int32),
            pltpu.VMEM((CK,), jnp.float32),
            pltpu.VMEM((CK,), jnp.float32),
            pltpu.VMEM((CK,), jnp.float32),
            pltpu.VMEM((CK,), jnp.float32),
            pltpu.VMEM_SHARED((ACC,), jnp.float32),
            pltpu.SemaphoreType.DMA,
            pltpu.SemaphoreType.DMA,
            pltpu.SemaphoreType.DMA,
            pltpu.SemaphoreType.DMA,
            pltpu.SemaphoreType.DMA,
        ],
    )


def _adj_matmul(adj_ref, m):
    mp = jnp.concatenate(
        [m, jnp.zeros((CT * 128 - NP, m.shape[1]), m.dtype)],
        axis=0).astype(jnp.bfloat16)
    acc = None
    for tc in range(CT):
        part = jnp.dot(adj_ref[pl.ds(tc * NP, NP), :],
                       mp[tc * 128:(tc + 1) * 128],
                       preferred_element_type=jnp.float32)
        acc = part if acc is None else acc + part
    return acc


def _conv1_body(x_ref, adj_ref, wpre_ref, bpre_ref, wrel_ref, brel_ref,
                wroot_ref, h1_ref, abf_ref, sum_ref, sq_ref, d_ref):
    xb = x_ref[0]
    h0 = jnp.dot(xb, wpre_ref[...],
                 preferred_element_type=jnp.float32) + bpre_ref[...]
    mp = jnp.concatenate(
        [h0, jnp.zeros((CT * 128 - NP, H), h0.dtype)],
        axis=0).astype(jnp.bfloat16)
    agg = None
    dparts = []
    for tc in range(CT):
        a32 = adj_ref[pl.ds(tc * NP, NP), :]
        abf = a32.astype(jnp.bfloat16)
        abf_ref[pl.ds(tc * NP, NP), :] = abf
        part = jnp.dot(abf, mp[tc * 128:(tc + 1) * 128],
                       preferred_element_type=jnp.float32)
        agg = part if agg is None else agg + part
        dparts.append(jnp.sum(a32, axis=0))
    h1 = (jnp.dot(agg, wrel_ref[...], preferred_element_type=jnp.float32)
          + jnp.dot(h0, wroot_ref[...], preferred_element_type=jnp.float32)
          + brel_ref[...])
    h1_ref[0] = h1
    sum_ref[0] = jnp.sum(h1, axis=0, keepdims=True)
    sq_ref[0] = jnp.sum(h1 * h1, axis=0, keepdims=True)
    d_ref[0] = jnp.concatenate(dparts)[:NP].reshape(1, NP)


def _bn_stats(sum_ref, sq_ref):
    mu = jnp.sum(sum_ref[...], axis=0) / (B * NP)
    ex2 = jnp.sum(sq_ref[...], axis=0) / (B * NP)
    var = ex2 - mu * mu
    return mu, lax.rsqrt(var + 1e-5)


def _conv2_body(h1_ref, adj_ref, sum_ref, sq_ref, g_ref, be_ref, wrel_ref,
                brel_ref, wroot_ref, h2_ref, sum2_ref, sq2_ref):
    mu, inv = _bn_stats(sum_ref, sq_ref)
    hn = jnp.maximum((h1_ref[0] - mu) * inv * g_ref[...] + be_ref[...], 0.0)
    agg = _adj_matmul(adj_ref, hn)
    h2 = (jnp.dot(agg, wrel_ref[...], preferred_element_type=jnp.float32)
          + jnp.dot(hn, wroot_ref[...], preferred_element_type=jnp.float32)
          + brel_ref[...])
    h2_ref[0] = h2
    sum2_ref[0] = jnp.sum(h2, axis=0, keepdims=True)
    sq2_ref[0] = jnp.sum(h2 * h2, axis=0, keepdims=True)


def _pool_body(h2_ref, adj_ref, d_ref, sum_ref, sq_ref, g_ref, be_ref,
               wpool_ref, bpool_ref, wpt_ref, wpb_ref, bpost_ref,
               out_ref, mc_ref, ol_ref):
    b = pl.program_id(0)
    mu, inv = _bn_stats(sum_ref, sq_ref)
    hn = jnp.maximum((h2_ref[0] - mu) * inv * g_ref[...] + be_ref[...], 0.0)
    z = jnp.dot(hn, wpool_ref[...],
                preferred_element_type=jnp.float32) + bpool_ref[...]
    m = jnp.max(z, axis=1, keepdims=True)
    ez = jnp.exp(z - m)
    s = ez / jnp.sum(ez, axis=1, keepdims=True)
    p0 = jnp.maximum(jnp.sum(s[:, 0:1] * hn, axis=0, keepdims=True), 0.0)
    p1 = jnp.maximum(jnp.sum(s[:, 1:2] * hn, axis=0, keepdims=True), 0.0)
    out = (jnp.dot(p0, wpt_ref[...], preferred_element_type=jnp.float32)
           + jnp.dot(p1, wpb_ref[...], preferred_element_type=jnp.float32)
           + bpost_ref[...])
    out_ref[0] = out
    w = _adj_matmul(adj_ref, s)
    num = jnp.sum(w * s)
    den = jnp.sum(d_ref[0, 0] * jnp.sum(s * s, axis=1))
    s0 = s[:, 0:1]
    s1 = s[:, 1:2]
    s00 = jnp.sum(s0 * s0)
    s01 = jnp.sum(s0 * s1)
    s11 = jnp.sum(s1 * s1)
    fro = jnp.sqrt(s00 * s00 + 2.0 * s01 * s01 + s11 * s11)
    c = 1.0 / jnp.sqrt(jnp.float32(2.0))
    a00 = s00 / fro - c
    a01 = s01 / fro
    a11 = s11 / fro - c
    mc = -(num / den) * (1.0 / B)
    ob = jnp.sqrt(a00 * a00 + 2.0 * a01 * a01 + a11 * a11) * (1.0 / B)

    @pl.when(b == 0)
    def _():
        mc_ref[...] = jnp.zeros((1, 1), jnp.float32)
        ol_ref[...] = jnp.zeros((1, 1), jnp.float32)
    mc_ref[...] += jnp.reshape(mc, (1, 1))
    ol_ref[...] += jnp.reshape(ob, (1, 1))


def _full(shape):
    return pl.BlockSpec(shape, lambda b: tuple(0 for _ in shape))


_conv1 = pl.pallas_call(
    _conv1_body,
    grid=(B,),
    in_specs=[
        pl.BlockSpec((1, NP, H), lambda b: (b, 0, 0)),
        pl.BlockSpec((CT * NP, 128), lambda b: (b, 0)),
        _full((H, H)), _full((1, H)), _full((H, H)), _full((1, H)),
        _full((H, H)),
    ],
    out_specs=[
        pl.BlockSpec((1, NP, H), lambda b: (b, 0, 0)),
        pl.BlockSpec((CT * NP, 128), lambda b: (b, 0)),
        pl.BlockSpec((1, 1, H), lambda b: (b, 0, 0)),
        pl.BlockSpec((1, 1, H), lambda b: (b, 0, 0)),
        pl.BlockSpec((1, 1, NP), lambda b: (b, 0, 0)),
    ],
    out_shape=[
        jax.ShapeDtypeStruct((B, NP, H), jnp.float32),
        jax.ShapeDtypeStruct((B * CT * NP, 128), jnp.bfloat16),
        jax.ShapeDtypeStruct((B, 1, H), jnp.float32),
        jax.ShapeDtypeStruct((B, 1, H), jnp.float32),
        jax.ShapeDtypeStruct((B, 1, NP), jnp.float32),
    ],
)

_conv2 = pl.pallas_call(
    _conv2_body,
    grid=(B,),
    in_specs=[
        pl.BlockSpec((1, NP, H), lambda b: (b, 0, 0)),
        pl.BlockSpec((CT * NP, 128), lambda b: (b, 0)),
        pl.BlockSpec((B, 1, H), lambda b: (0, 0, 0)),
        pl.BlockSpec((B, 1, H), lambda b: (0, 0, 0)),
        _full((1, H)), _full((1, H)), _full((H, H)), _full((1, H)),
        _full((H, H)),
    ],
    out_specs=[
        pl.BlockSpec((1, NP, H), lambda b: (b, 0, 0)),
        pl.BlockSpec((1, 1, H), lambda b: (b, 0, 0)),
        pl.BlockSpec((1, 1, H), lambda b: (b, 0, 0)),
    ],
    out_shape=[
        jax.ShapeDtypeStruct((B, NP, H), jnp.float32),
        jax.ShapeDtypeStruct((B, 1, H), jnp.float32),
        jax.ShapeDtypeStruct((B, 1, H), jnp.float32),
    ],
)

_pool = pl.pallas_call(
    _pool_body,
    grid=(B,),
    in_specs=[
        pl.BlockSpec((1, NP, H), lambda b: (b, 0, 0)),
        pl.BlockSpec((CT * NP, 128), lambda b: (b, 0)),
        pl.BlockSpec((1, 1, NP), lambda b: (b, 0, 0)),
        pl.BlockSpec((B, 1, H), lambda b: (0, 0, 0)),
        pl.BlockSpec((B, 1, H), lambda b: (0, 0, 0)),
        _full((1, H)), _full((1, H)), _full((H, 2)), _full((1, 2)),
        _full((H, 2)), _full((H, 2)), _full((1, 2)),
    ],
    out_specs=[
        pl.BlockSpec((1, 1, 2), lambda b: (b, 0, 0)),
        pl.BlockSpec((1, 1), lambda b: (0, 0)),
        pl.BlockSpec((1, 1), lambda b: (0, 0)),
    ],
    out_shape=[
        jax.ShapeDtypeStruct((B, 1, 2), jnp.float32),
        jax.ShapeDtypeStruct((1, 1), jnp.float32),
        jax.ShapeDtypeStruct((1, 1), jnp.float32),
    ],
)


def kernel(x, edge_index, edge_feature, batch, W_pre, b_pre, Wrel1, brel1,
           Wroot1, g1, be1, Wrel2, brel2, Wroot2, g2, be2, W_pool, b_pool,
           W_post, b_post):
    ew = edge_feature.reshape(E)
    adj = _build_adj_kernel()(edge_index.reshape(2 * E),
                              ew).reshape(B * CT * NP, 128)
    x3 = x.reshape(B, NP, H)
    h1, abf, s1, q1, d = _conv1(x3, adj, W_pre, b_pre.reshape(1, H), Wrel1,
                                brel1.reshape(1, H), Wroot1)
    h2, s2, q2 = _conv2(h1, abf, s1, q1, g1.reshape(1, H), be1.reshape(1, H),
                        Wrel2, brel2.reshape(1, H), Wroot2)
    out3, mc, ol = _pool(h2, abf, d, s2, q2, g2.reshape(1, H),
                         be2.reshape(1, H), W_pool, b_pool.reshape(1, 2),
                         W_post[:H], W_post[H:], b_post.reshape(1, 2))
    return out3.reshape(B, 2), mc[0, 0], ol[0, 0]
